# baseline (speedup 1.0000x reference)
import functools

import jax
import jax.numpy as jnp
from jax import lax
from jax.experimental import pallas as pl
from jax.experimental.pallas import tpu as pltpu
from jax.experimental.pallas import tpu_sc as plsc

N = 10000
E = 320000
D = 128
H = 64
G = 64

NC = 2
NS = 16
NW = NC * NS
CH = 128
KCH = 80
EPAD = NW * KCH * CH
NPAD = 10240
RPT = NPAD // NS

BN = 1000
GRID = N // BN


def _mm_body(x_ref, w_ref, o_ref):
    o_ref[...] = jnp.dot(x_ref[...], w_ref[...],
                         preferred_element_type=jnp.float32)


def _matmul(x, w):
    n, d = x.shape
    h = w.shape[1]
    return pl.pallas_call(
        _mm_body,
        grid=(n // BN,),
        in_specs=[pl.BlockSpec((BN, d), lambda i: (i, 0)),
                  pl.BlockSpec((d, h), lambda i: (0, 0))],
        out_specs=pl.BlockSpec((BN, h), lambda i: (i, 0)),
        out_shape=jax.ShapeDtypeStruct((n, h), jnp.float32),
    )(x, w)


def _sc_agg_body(u_hbm, src_hbm, dst_hbm, zeros_hbm, out_hbm,
                 src_v, dst_v, msg_a, msg_b, acc, sem_a, sem_b):
    c = lax.axis_index("c")
    s = lax.axis_index("s")
    wid = s * NC + c

    pltpu.sync_copy(zeros_hbm, acc.at[pl.ds(s * RPT, RPT)])
    pltpu.sync_copy(src_hbm.at[wid], src_v)
    pltpu.sync_copy(dst_hbm.at[wid], dst_v)
    plsc.subcore_barrier()

    pltpu.async_copy(u_hbm.at[src_v.at[0]], msg_a, sem_a)

    def body(jj, carry):
        j0 = 2 * jj
        pltpu.make_async_copy(u_hbm.at[src_v.at[j0]], msg_a, sem_a).wait()
        pltpu.async_copy(u_hbm.at[src_v.at[j0 + 1]], msg_b, sem_b)
        pltpu.sync_copy(msg_a, acc.at[dst_v.at[j0]], add=True)
        pltpu.make_async_copy(u_hbm.at[src_v.at[j0 + 1]], msg_b, sem_b).wait()

        @pl.when(jj + 1 < KCH // 2)
        def _():
            pltpu.async_copy(u_hbm.at[src_v.at[j0 + 2]], msg_a, sem_a)

        pltpu.sync_copy(msg_b, acc.at[dst_v.at[j0 + 1]], add=True)
        return carry

    lax.fori_loop(0, KCH // 2, body, 0)
    plsc.subcore_barrier()

    pltpu.sync_copy(acc.at[pl.ds(s * RPT, RPT)],
                    out_hbm.at[c, pl.ds(s * RPT, RPT)])


_sc_agg_cache = []


def _sc_agg(u, src_r, dst_r, zeros):
    if not _sc_agg_cache:
        _sc_agg_cache.append(pl.kernel(
            _sc_agg_body,
            out_type=jax.ShapeDtypeStruct((NC, NPAD, H), jnp.float32),
            mesh=plsc.VectorSubcoreMesh(core_axis_name="c",
                                        subcore_axis_name="s",
                                        num_cores=NC, num_subcores=NS),
            scratch_types=[
                pltpu.VMEM((KCH, CH), jnp.int32),
                pltpu.VMEM((KCH, CH), jnp.int32),
                pltpu.VMEM((CH, H), jnp.float32),
                pltpu.VMEM((CH, H), jnp.float32),
                pltpu.VMEM_SHARED((NPAD, H), jnp.float32),
                pltpu.SemaphoreType.DMA,
                pltpu.SemaphoreType.DMA,
            ],
            compiler_params=pltpu.CompilerParams(use_tc_tiling_on_sc=False),
        ))
    return _sc_agg_cache[0](u, src_r, dst_r, zeros)


def _mlp_body(u_ref, acc_ref, ba_ref, wb_ref, bb_ref, t_ref, s_ref):
    i = pl.program_id(0)
    z = u_ref[...] + acc_ref[0] + acc_ref[1] + ba_ref[...]
    z = jnp.maximum(z, 0.0)
    t = jnp.dot(z, wb_ref[...], preferred_element_type=jnp.float32) + bb_ref[...]
    t_ref[...] = t
    sq = jnp.concatenate([jnp.sum(t, 0, keepdims=True),
                          jnp.sum(t * t, 0, keepdims=True)], 0)

    @pl.when(i == 0)
    def _():
        s_ref[...] = sq

    @pl.when(i > 0)
    def _():
        s_ref[...] = s_ref[...] + sq


def _mlp_and_sums(u, accp, ba, wb, bb):
    return pl.pallas_call(
        _mlp_body,
        grid=(GRID,),
        in_specs=[pl.BlockSpec((BN, H), lambda i: (i, 0)),
                  pl.BlockSpec((NC, BN, H), lambda i: (0, i, 0)),
                  pl.BlockSpec((1, H), lambda i: (0, 0)),
                  pl.BlockSpec((H, H), lambda i: (0, 0)),
                  pl.BlockSpec((1, H), lambda i: (0, 0))],
        out_specs=[pl.BlockSpec((BN, H), lambda i: (i, 0)),
                   pl.BlockSpec((2, H), lambda i: (0, 0))],
        out_shape=[jax.ShapeDtypeStruct((N, H), jnp.float32),
                   jax.ShapeDtypeStruct((2, H), jnp.float32)],
    )(u, accp, ba, wb, bb)


def _bnmm_body(t_ref, s_ref, g_ref, be_ref, w_ref, o_ref):
    mu = s_ref[0:1] * (1.0 / N)
    var = s_ref[1:2] * (1.0 / N) - mu * mu
    a = g_ref[...] * lax.rsqrt(var + 1e-5)
    cc = be_ref[...] - mu * a
    h = jnp.maximum(t_ref[...] * a + cc, 0.0)
    o_ref[...] = jnp.dot(h, w_ref[...], preferred_element_type=jnp.float32)


def _bn_relu_matmul(t, sums, g, be, w):
    return pl.pallas_call(
        _bnmm_body,
        grid=(GRID,),
        in_specs=[pl.BlockSpec((BN, H), lambda i: (i, 0)),
                  pl.BlockSpec((2, H), lambda i: (0, 0)),
                  pl.BlockSpec((1, H), lambda i: (0, 0)),
                  pl.BlockSpec((1, H), lambda i: (0, 0)),
                  pl.BlockSpec((H, H), lambda i: (0, 0))],
        out_specs=pl.BlockSpec((BN, H), lambda i: (i, 0)),
        out_shape=jax.ShapeDtypeStruct((N, H), jnp.float32),
    )(t, sums, g, be, w)


def _pool_body(t_ref, s_ref, g_ref, be_ref, b_ref, o_ref, acc_scr, cnt_scr):
    i = pl.program_id(0)
    mu = s_ref[0:1] * (1.0 / N)
    var = s_ref[1:2] * (1.0 / N) - mu * mu
    a = g_ref[...] * lax.rsqrt(var + 1e-5)
    cc = be_ref[...] - mu * a
    h = jnp.maximum(t_ref[...] * a + cc, 0.0)

    lbl = b_ref[0]
    oh = (lbl == lax.broadcasted_iota(jnp.int32, (G, BN), 0))
    oh = oh.astype(jnp.float32)
    pooled = jnp.dot(oh, h, preferred_element_type=jnp.float32)
    cnt = jnp.sum(oh, 1, keepdims=True)

    @pl.when(i == 0)
    def _():
        acc_scr[...] = pooled
        cnt_scr[...] = cnt

    @pl.when(i > 0)
    def _():
        acc_scr[...] = acc_scr[...] + pooled
        cnt_scr[...] = cnt_scr[...] + cnt

    @pl.when(i == GRID - 1)
    def _():
        o_ref[...] = acc_scr[...] / jnp.maximum(cnt_scr[...], 1.0)


def _bn_relu_pool(t, sums, g, be, batch_r):
    return pl.pallas_call(
        _pool_body,
        grid=(GRID,),
        in_specs=[pl.BlockSpec((BN, H), lambda i: (i, 0)),
                  pl.BlockSpec((2, H), lambda i: (0, 0)),
                  pl.BlockSpec((1, H), lambda i: (0, 0)),
                  pl.BlockSpec((1, H), lambda i: (0, 0)),
                  pl.BlockSpec((1, 1, BN), lambda i: (i, 0, 0))],
        out_specs=pl.BlockSpec((G, H), lambda i: (0, 0)),
        out_shape=jax.ShapeDtypeStruct((G, H), jnp.float32),
        scratch_shapes=[pltpu.VMEM((G, H), jnp.float32),
                        pltpu.VMEM((G, 1), jnp.float32)],
    )(t, sums, g, be, batch_r)


def kernel(x, ei, batch, W1a, b1a, W1b, b1b, g1, be1,
           W2a, b2a, W2b, b2b, g2, be2):
    pad = EPAD - E
    src = jnp.concatenate([ei[0], jnp.zeros((pad,), jnp.int32)])
    dst_pad = N + (jnp.arange(pad, dtype=jnp.int32) % (NPAD - N))
    dst = jnp.concatenate([ei[1], dst_pad])
    src_r = src.reshape(NW, KCH, CH)
    dst_r = dst.reshape(NW, KCH, CH)
    zeros = jnp.zeros((RPT, H), jnp.float32)
    batch_r = batch.reshape(GRID, 1, BN)
    b1a_r, b1b_r = b1a.reshape(1, H), b1b.reshape(1, H)
    b2a_r, b2b_r = b2a.reshape(1, H), b2b.reshape(1, H)
    g1_r, be1_r = g1.reshape(1, H), be1.reshape(1, H)
    g2_r, be2_r = g2.reshape(1, H), be2.reshape(1, H)

    u1 = _matmul(x, W1a)
    accp1 = _sc_agg(u1, src_r, dst_r, zeros)
    t1, s1 = _mlp_and_sums(u1, accp1, b1a_r, W1b, b1b_r)

    u2 = _bn_relu_matmul(t1, s1, g1_r, be1_r, W2a)
    accp2 = _sc_agg(u2, src_r, dst_r, zeros)
    t2, s2 = _mlp_and_sums(u2, accp2, b2a_r, W2b, b2b_r)

    return _bn_relu_pool(t2, s2, g2_r, be2_r, batch_r)


# speedup vs baseline: 1.1175x; 1.1175x over previous
import functools

import jax
import jax.numpy as jnp
from jax import lax
from jax.experimental import pallas as pl
from jax.experimental.pallas import tpu as pltpu
from jax.experimental.pallas import tpu_sc as plsc

N = 10000
E = 320000
D = 128
H = 64
G = 64

NC = 2
NS = 16
NW = NC * NS
CH = 128
KCH = 80
EPAD = NW * KCH * CH
NPAD = 10240
RPT = NPAD // NS

BN = 1000
GRID = N // BN


def _mm_body(x_ref, w_ref, o_ref):
    o_ref[...] = jnp.dot(x_ref[...], w_ref[...],
                         preferred_element_type=jnp.float32)


def _matmul(x, w):
    n, d = x.shape
    h = w.shape[1]
    return pl.pallas_call(
        _mm_body,
        grid=(n // BN,),
        in_specs=[pl.BlockSpec((BN, d), lambda i: (i, 0)),
                  pl.BlockSpec((d, h), lambda i: (0, 0))],
        out_specs=pl.BlockSpec((BN, h), lambda i: (i, 0)),
        out_shape=jax.ShapeDtypeStruct((n, h), jnp.float32),
    )(x, w)


def _sc_agg_body(u_hbm, src_hbm, dst_hbm, zeros_hbm, out_hbm,
                 src_v, dst_v, msg_a, msg_b, msg_c, msg_d, acc,
                 gsem_a, gsem_b, gsem_c, gsem_d,
                 ssem_a, ssem_b, ssem_c, ssem_d):
    c = lax.axis_index("c")
    s = lax.axis_index("s")
    wid = s * NC + c

    pltpu.sync_copy(zeros_hbm, acc.at[pl.ds(s * RPT, RPT)])
    pltpu.sync_copy(src_hbm.at[wid], src_v)
    pltpu.sync_copy(dst_hbm.at[wid], dst_v)
    plsc.subcore_barrier()

    msg = (msg_a, msg_b, msg_c, msg_d)
    gsem = (gsem_a, gsem_b, gsem_c, gsem_d)
    ssem = (ssem_a, ssem_b, ssem_c, ssem_d)

    pltpu.async_copy(u_hbm.at[src_v.at[0]], msg[0], gsem[0])
    pltpu.async_copy(u_hbm.at[src_v.at[1]], msg[1], gsem[1])

    def body(jj, carry):
        for b in range(4):
            j = 4 * jj + b
            bg = (b + 2) % 4

            @pl.when(j >= 2)
            def _():
                pltpu.make_async_copy(msg[bg], acc.at[dst_v.at[j - 2]],
                                      ssem[bg]).wait()

            @pl.when(j + 2 < KCH)
            def _():
                pltpu.async_copy(u_hbm.at[src_v.at[j + 2]], msg[bg], gsem[bg])

            pltpu.make_async_copy(u_hbm.at[src_v.at[j]], msg[b], gsem[b]).wait()
            pltpu.async_copy(msg[b], acc.at[dst_v.at[j]], ssem[b], add=True)
        return carry

    lax.fori_loop(0, KCH // 4, body, 0)
    pltpu.make_async_copy(msg[2], acc.at[dst_v.at[KCH - 2]], ssem[2]).wait()
    pltpu.make_async_copy(msg[3], acc.at[dst_v.at[KCH - 1]], ssem[3]).wait()
    plsc.subcore_barrier()

    pltpu.sync_copy(acc.at[pl.ds(s * RPT, RPT)],
                    out_hbm.at[c, pl.ds(s * RPT, RPT)])


_sc_agg_cache = []


def _sc_agg(u, src_r, dst_r, zeros):
    if not _sc_agg_cache:
        _sc_agg_cache.append(pl.kernel(
            _sc_agg_body,
            out_type=jax.ShapeDtypeStruct((NC, NPAD, H), jnp.float32),
            mesh=plsc.VectorSubcoreMesh(core_axis_name="c",
                                        subcore_axis_name="s",
                                        num_cores=NC, num_subcores=NS),
            scratch_types=(
                [pltpu.VMEM((KCH, CH), jnp.int32),
                 pltpu.VMEM((KCH, CH), jnp.int32)]
                + [pltpu.VMEM((CH, H), jnp.float32) for _ in range(4)]
                + [pltpu.VMEM_SHARED((NPAD, H), jnp.float32)]
                + [pltpu.SemaphoreType.DMA for _ in range(8)]
            ),
            compiler_params=pltpu.CompilerParams(use_tc_tiling_on_sc=False),
        ))
    return _sc_agg_cache[0](u, src_r, dst_r, zeros)


def _mlp_body(u_ref, acc_ref, ba_ref, wb_ref, bb_ref, t_ref, s_ref):
    i = pl.program_id(0)
    z = u_ref[...] + acc_ref[0] + acc_ref[1] + ba_ref[...]
    z = jnp.maximum(z, 0.0)
    t = jnp.dot(z, wb_ref[...], preferred_element_type=jnp.float32) + bb_ref[...]
    t_ref[...] = t
    sq = jnp.concatenate([jnp.sum(t, 0, keepdims=True),
                          jnp.sum(t * t, 0, keepdims=True)], 0)

    @pl.when(i == 0)
    def _():
        s_ref[...] = sq

    @pl.when(i > 0)
    def _():
        s_ref[...] = s_ref[...] + sq


def _mlp_and_sums(u, accp, ba, wb, bb):
    return pl.pallas_call(
        _mlp_body,
        grid=(GRID,),
        in_specs=[pl.BlockSpec((BN, H), lambda i: (i, 0)),
                  pl.BlockSpec((NC, BN, H), lambda i: (0, i, 0)),
                  pl.BlockSpec((1, H), lambda i: (0, 0)),
                  pl.BlockSpec((H, H), lambda i: (0, 0)),
                  pl.BlockSpec((1, H), lambda i: (0, 0))],
        out_specs=[pl.BlockSpec((BN, H), lambda i: (i, 0)),
                   pl.BlockSpec((2, H), lambda i: (0, 0))],
        out_shape=[jax.ShapeDtypeStruct((N, H), jnp.float32),
                   jax.ShapeDtypeStruct((2, H), jnp.float32)],
    )(u, accp, ba, wb, bb)


def _bnmm_body(t_ref, s_ref, g_ref, be_ref, w_ref, o_ref):
    mu = s_ref[0:1] * (1.0 / N)
    var = s_ref[1:2] * (1.0 / N) - mu * mu
    a = g_ref[...] * lax.rsqrt(var + 1e-5)
    cc = be_ref[...] - mu * a
    h = jnp.maximum(t_ref[...] * a + cc, 0.0)
    o_ref[...] = jnp.dot(h, w_ref[...], preferred_element_type=jnp.float32)


def _bn_relu_matmul(t, sums, g, be, w):
    return pl.pallas_call(
        _bnmm_body,
        grid=(GRID,),
        in_specs=[pl.BlockSpec((BN, H), lambda i: (i, 0)),
                  pl.BlockSpec((2, H), lambda i: (0, 0)),
                  pl.BlockSpec((1, H), lambda i: (0, 0)),
                  pl.BlockSpec((1, H), lambda i: (0, 0)),
                  pl.BlockSpec((H, H), lambda i: (0, 0))],
        out_specs=pl.BlockSpec((BN, H), lambda i: (i, 0)),
        out_shape=jax.ShapeDtypeStruct((N, H), jnp.float32),
    )(t, sums, g, be, w)


def _pool_body(t_ref, s_ref, g_ref, be_ref, b_ref, o_ref, acc_scr, cnt_scr):
    i = pl.program_id(0)
    mu = s_ref[0:1] * (1.0 / N)
    var = s_ref[1:2] * (1.0 / N) - mu * mu
    a = g_ref[...] * lax.rsqrt(var + 1e-5)
    cc = be_ref[...] - mu * a
    h = jnp.maximum(t_ref[...] * a + cc, 0.0)

    lbl = b_ref[0]
    oh = (lbl == lax.broadcasted_iota(jnp.int32, (G, BN), 0))
    oh = oh.astype(jnp.float32)
    pooled = jnp.dot(oh, h, preferred_element_type=jnp.float32)
    cnt = jnp.sum(oh, 1, keepdims=True)

    @pl.when(i == 0)
    def _():
        acc_scr[...] = pooled
        cnt_scr[...] = cnt

    @pl.when(i > 0)
    def _():
        acc_scr[...] = acc_scr[...] + pooled
        cnt_scr[...] = cnt_scr[...] + cnt

    @pl.when(i == GRID - 1)
    def _():
        o_ref[...] = acc_scr[...] / jnp.maximum(cnt_scr[...], 1.0)


def _bn_relu_pool(t, sums, g, be, batch_r):
    return pl.pallas_call(
        _pool_body,
        grid=(GRID,),
        in_specs=[pl.BlockSpec((BN, H), lambda i: (i, 0)),
                  pl.BlockSpec((2, H), lambda i: (0, 0)),
                  pl.BlockSpec((1, H), lambda i: (0, 0)),
                  pl.BlockSpec((1, H), lambda i: (0, 0)),
                  pl.BlockSpec((1, 1, BN), lambda i: (i, 0, 0))],
        out_specs=pl.BlockSpec((G, H), lambda i: (0, 0)),
        out_shape=jax.ShapeDtypeStruct((G, H), jnp.float32),
        scratch_shapes=[pltpu.VMEM((G, H), jnp.float32),
                        pltpu.VMEM((G, 1), jnp.float32)],
    )(t, sums, g, be, batch_r)


def kernel(x, ei, batch, W1a, b1a, W1b, b1b, g1, be1,
           W2a, b2a, W2b, b2b, g2, be2):
    pad = EPAD - E
    src = jnp.concatenate([ei[0], jnp.zeros((pad,), jnp.int32)])
    dst_pad = N + (jnp.arange(pad, dtype=jnp.int32) % (NPAD - N))
    dst = jnp.concatenate([ei[1], dst_pad])
    src_r = src.reshape(NW, KCH, CH)
    dst_r = dst.reshape(NW, KCH, CH)
    zeros = jnp.zeros((RPT, H), jnp.float32)
    batch_r = batch.reshape(GRID, 1, BN)
    b1a_r, b1b_r = b1a.reshape(1, H), b1b.reshape(1, H)
    b2a_r, b2b_r = b2a.reshape(1, H), b2b.reshape(1, H)
    g1_r, be1_r = g1.reshape(1, H), be1.reshape(1, H)
    g2_r, be2_r = g2.reshape(1, H), be2.reshape(1, H)

    u1 = _matmul(x, W1a)
    accp1 = _sc_agg(u1, src_r, dst_r, zeros)
    t1, s1 = _mlp_and_sums(u1, accp1, b1a_r, W1b, b1b_r)

    u2 = _bn_relu_matmul(t1, s1, g1_r, be1_r, W2a)
    accp2 = _sc_agg(u2, src_r, dst_r, zeros)
    t2, s2 = _mlp_and_sums(u2, accp2, b2a_r, W2b, b2b_r)

    return _bn_relu_pool(t2, s2, g2_r, be2_r, batch_r)
